# baseline (speedup 1.0000x reference)
import functools

import jax
import jax.numpy as jnp
from jax import lax
from jax.experimental import pallas as pl
from jax.experimental.pallas import tpu as pltpu
from jax.experimental.pallas import tpu_sc as plsc

_L = 16


def _sc_info():
    try:
        info = plsc.get_sparse_core_info()
        return info.num_cores, info.num_subcores
    except Exception:
        return 2, 16


def _segmax_t_body(nw, w, fpt, e, c, ei_hbm, ht_hbm, out_hbm,
                   hts, acc, tmpa, tmpb, sa, da, sb, db, sema, semb):
    nc, _ = _sc_info()
    wid = lax.axis_index("s") * nc + lax.axis_index("c")
    slab = fpt * w
    base = wid * slab
    neg = jnp.full((_L,), -jnp.inf, jnp.float32)
    lane = lax.iota(jnp.int32, _L)
    nch = e // c

    pltpu.sync_copy(ht_hbm.at[pl.ds(base, slab)], hts)

    def init_chunk(i, _):
        acc[pl.ds(i * _L, _L)] = neg
        return 0

    lax.fori_loop(0, slab // _L, init_chunk, 0)

    def start(ci, s_buf, d_buf, sem):
        pltpu.async_copy(ei_hbm.at[0, pl.ds(ci * c, c)], s_buf, sem)
        pltpu.async_copy(ei_hbm.at[1, pl.ds(ci * c, c)], d_buf, sem)

    def wait(s_buf, d_buf, sem):
        pltpu.make_async_copy(ei_hbm.at[0, pl.ds(0, c)], s_buf, sem).wait()
        pltpu.make_async_copy(ei_hbm.at[0, pl.ds(0, c)], d_buf, sem).wait()

    def process(s_buf, d_buf):
        unroll = 4

        def round_(s16, d16, hvs, rem, tmp):
            idxw = jnp.where(rem, d16, w)
            plsc.store_scatter(tmp, [idxw], lane)
            back = plsc.load_gather(tmp, [idxw])
            winner = rem & (back == lane)
            idxus = [jnp.where(winner, d16 + f * w, slab) for f in range(fpt)]
            avs = [plsc.load_gather(acc, [idxus[f]]) for f in range(fpt)]
            mxs = [jnp.maximum(avs[f], hvs[f]) for f in range(fpt)]
            for f in range(fpt):
                plsc.store_scatter(acc, [idxus[f]], mxs[f])
            return rem & (~winner)

        def group4(q, _):
            s16s = [s_buf[pl.ds((q * unroll + u) * _L, _L)]
                    for u in range(unroll)]
            d16s = [d_buf[pl.ds((q * unroll + u) * _L, _L)]
                    for u in range(unroll)]
            hvss = [[plsc.load_gather(hts, [s16s[u] + f * w])
                     for f in range(fpt)] for u in range(unroll)]
            rems = []
            for u in range(unroll):
                rems.append(round_(s16s[u], d16s[u], hvss[u],
                                   d16s[u] >= 0,
                                   tmpa if u % 2 == 0 else tmpb))

            anyrem = rems[0]
            for u in range(1, unroll):
                anyrem = anyrem | rems[u]

            @pl.when(jnp.any(anyrem))
            def _():
                for u in range(unroll):
                    gb = (q * unroll + u) * _L
                    s16 = s_buf[pl.ds(gb, _L)]
                    d16 = d_buf[pl.ds(gb, _L)]
                    hvs = [plsc.load_gather(hts, [s16 + f * w])
                           for f in range(fpt)]

                    def cond(carry):
                        rem, r = carry
                        return jnp.any(rem) & (r < _L)

                    def wbody(carry):
                        rem, r = carry
                        return round_(s16, d16, hvs, rem, tmpa), r + 1

                    lax.while_loop(cond, wbody, (d16 >= 0, 0))

            return 0

        lax.fori_loop(0, c // (_L * unroll), group4, 0)

    start(0, sa, da, sema)

    def pairbody(p, _):
        start(2 * p + 1, sb, db, semb)
        wait(sa, da, sema)
        process(sa, da)
        start(2 * p + 2, sa, da, sema)
        wait(sb, db, semb)
        process(sb, db)
        return 0

    lax.fori_loop(0, (nch - 1) // 2, pairbody, 0)
    wait(sa, da, sema)
    process(sa, da)

    def fix_chunk(i, _):
        a = acc[pl.ds(i * _L, _L)]
        acc[pl.ds(i * _L, _L)] = jnp.where(a == neg, 0.0, a)
        return 0

    lax.fori_loop(0, slab // _L, fix_chunk, 0)
    pltpu.sync_copy(acc.at[pl.ds(0, slab)], out_hbm.at[pl.ds(base, slab)])


def _transpose_tc(x):
    def body(x_ref, o_ref):
        o_ref[...] = x_ref[...].T

    return pl.pallas_call(
        body,
        out_shape=jax.ShapeDtypeStruct((x.shape[1], x.shape[0]), jnp.float32),
    )(x)


def _layer_tc(aggT, hT, wl, bl, wr):
    def body(a_ref, h_ref, wl_ref, bl_ref, wr_ref, o_ref):
        o_ref[...] = (
            jnp.dot(wl_ref[...], a_ref[...], preferred_element_type=jnp.float32)
            + jnp.dot(wr_ref[...], h_ref[...], preferred_element_type=jnp.float32)
            + bl_ref[...])

    return pl.pallas_call(
        body, out_shape=jax.ShapeDtypeStruct(aggT.shape, jnp.float32),
    )(aggT, hT, wl, bl.reshape(-1, 1), wr)


def _head_tc(aggT, hT, wl, bl, wr, w1, b1, w2, b2, w3p, b3p):
    def body(a_ref, h_ref, wl_ref, bl_ref, wr_ref, w1_ref, b1_ref, w2_ref,
             b2_ref, w3_ref, b3_ref, o_ref):
        h3 = (jnp.dot(wl_ref[...], a_ref[...], preferred_element_type=jnp.float32)
              + jnp.dot(wr_ref[...], h_ref[...], preferred_element_type=jnp.float32)
              + bl_ref[...])
        t = jnp.maximum(
            jnp.dot(w1_ref[...], h3, preferred_element_type=jnp.float32)
            + b1_ref[...], 0.0)
        t = jnp.maximum(
            jnp.dot(w2_ref[...], t, preferred_element_type=jnp.float32)
            + b2_ref[...], 0.0)
        t = (jnp.dot(w3_ref[...], t, preferred_element_type=jnp.float32)
             + b3_ref[...])
        o_ref[...] = 1.0 / (1.0 + jnp.exp(-t))

    return pl.pallas_call(
        body,
        out_shape=jax.ShapeDtypeStruct((w3p.shape[0], aggT.shape[1]),
                                       jnp.float32),
    )(aggT, hT, wl, bl.reshape(-1, 1), wr, w1, b1.reshape(-1, 1), w2,
      b2.reshape(-1, 1), w3p, b3p.reshape(-1, 1))


def kernel(x, edge_index, batch, W1l, b1l, W1r, W2l, b2l, W2r, W3l, b3l, W3r,
           l1W, l1b, l2W, l2b, l3W, l3b):
    n, d = x.shape
    e = edge_index.shape[1]
    nc, ns = _sc_info()
    nw = nc * ns
    fpt = d // nw

    c = 2560
    while e % c:
        c //= 2

    mesh = plsc.VectorSubcoreMesh(core_axis_name="c", subcore_axis_name="s")

    segmax = pl.kernel(
        functools.partial(_segmax_t_body, nw, n, fpt, e, c),
        out_type=jax.ShapeDtypeStruct((d * n,), jnp.float32),
        mesh=mesh,
        scratch_types=[
            pltpu.VMEM((fpt * n,), jnp.float32),
            pltpu.VMEM((fpt * n + _L,), jnp.float32),
            pltpu.VMEM((n + _L,), jnp.int32),
            pltpu.VMEM((n + _L,), jnp.int32),
            pltpu.VMEM((c,), jnp.int32),
            pltpu.VMEM((c,), jnp.int32),
            pltpu.VMEM((c,), jnp.int32),
            pltpu.VMEM((c,), jnp.int32),
            pltpu.SemaphoreType.DMA,
            pltpu.SemaphoreType.DMA,
        ],
        compiler_params=pltpu.CompilerParams(needs_layout_passes=False),
    )

    def agg_of(hT):
        return segmax(edge_index, hT.reshape(-1)).reshape(d, n)

    w3p = jnp.zeros((8, l3W.shape[1]), jnp.float32).at[:l3W.shape[0]].set(l3W)
    b3p = jnp.zeros((8,), jnp.float32).at[:l3b.shape[0]].set(l3b)

    xT = _transpose_tc(x)
    h1T = _layer_tc(agg_of(xT), xT, W1l, b1l, W1r)
    h2T = _layer_tc(agg_of(h1T), h1T, W2l, b2l, W2r)
    outT = _head_tc(agg_of(h2T), h2T, W3l, b3l, W3r, l1W, l1b, l2W, l2b,
                    w3p, b3p)
    return outT[:l3W.shape[0]].T


# speedup vs baseline: 1.0312x; 1.0312x over previous
import functools

import jax
import jax.numpy as jnp
from jax import lax
from jax.experimental import pallas as pl
from jax.experimental.pallas import tpu as pltpu
from jax.experimental.pallas import tpu_sc as plsc

_L = 16


def _sc_info():
    try:
        info = plsc.get_sparse_core_info()
        return info.num_cores, info.num_subcores
    except Exception:
        return 2, 16


def _segmax_t_body(nw, w, fpt, e, c, ei_hbm, ht_hbm, out_hbm,
                   hts, acc, tmpa, tmpb, sa, da, sb, db, sema, semb):
    nc, _ = _sc_info()
    wid = lax.axis_index("s") * nc + lax.axis_index("c")
    slab = fpt * w
    base = wid * slab
    neg = jnp.full((_L,), -jnp.inf, jnp.float32)
    lane = lax.iota(jnp.int32, _L)
    nch = e // c

    pltpu.sync_copy(ht_hbm.at[pl.ds(base, slab)], hts)

    def init_chunk(i, _):
        for u in range(4):
            acc[pl.ds((4 * i + u) * _L, _L)] = neg
        return 0

    lax.fori_loop(0, slab // (4 * _L), init_chunk, 0)

    def start(ci, s_buf, d_buf, sem):
        pltpu.async_copy(ei_hbm.at[0, pl.ds(ci * c, c)], s_buf, sem)
        pltpu.async_copy(ei_hbm.at[1, pl.ds(ci * c, c)], d_buf, sem)

    def wait(s_buf, d_buf, sem):
        pltpu.make_async_copy(ei_hbm.at[0, pl.ds(0, c)], s_buf, sem).wait()
        pltpu.make_async_copy(ei_hbm.at[0, pl.ds(0, c)], d_buf, sem).wait()

    def process(s_buf, d_buf):
        unroll = 4

        def round_(s16, d16, hvs, rem, tmp):
            idxw = jnp.where(rem, d16, w)
            plsc.store_scatter(tmp, [idxw], lane)
            back = plsc.load_gather(tmp, [idxw])
            winner = rem & (back == lane)
            idxus = [jnp.where(winner, d16 + f * w, slab) for f in range(fpt)]
            avs = [plsc.load_gather(acc, [idxus[f]]) for f in range(fpt)]
            mxs = [jnp.maximum(avs[f], hvs[f]) for f in range(fpt)]
            for f in range(fpt):
                plsc.store_scatter(acc, [idxus[f]], mxs[f])
            return rem & (~winner)

        def group4(q, _):
            s16s = [s_buf[pl.ds((q * unroll + u) * _L, _L)]
                    for u in range(unroll)]
            d16s = [d_buf[pl.ds((q * unroll + u) * _L, _L)]
                    for u in range(unroll)]
            hvss = [[plsc.load_gather(hts, [s16s[u] + f * w])
                     for f in range(fpt)] for u in range(unroll)]
            rems = []
            for u in range(unroll):
                rems.append(round_(s16s[u], d16s[u], hvss[u],
                                   d16s[u] >= 0,
                                   tmpa if u % 2 == 0 else tmpb))

            anyrem = rems[0]
            for u in range(1, unroll):
                anyrem = anyrem | rems[u]

            @pl.when(jnp.any(anyrem))
            def _():
                for u in range(unroll):
                    gb = (q * unroll + u) * _L
                    s16 = s_buf[pl.ds(gb, _L)]
                    d16 = d_buf[pl.ds(gb, _L)]
                    hvs = [plsc.load_gather(hts, [s16 + f * w])
                           for f in range(fpt)]

                    def cond(carry):
                        rem, r = carry
                        return jnp.any(rem) & (r < _L)

                    def wbody(carry):
                        rem, r = carry
                        return round_(s16, d16, hvs, rem, tmpa), r + 1

                    lax.while_loop(cond, wbody, (d16 >= 0, 0))

            return 0

        lax.fori_loop(0, c // (_L * unroll), group4, 0)

    start(0, sa, da, sema)

    def pairbody(p, _):
        start(2 * p + 1, sb, db, semb)
        wait(sa, da, sema)
        process(sa, da)
        start(2 * p + 2, sa, da, sema)
        wait(sb, db, semb)
        process(sb, db)
        return 0

    lax.fori_loop(0, (nch - 1) // 2, pairbody, 0)
    wait(sa, da, sema)
    process(sa, da)

    def fix_chunk(i, _):
        avs = [acc[pl.ds((4 * i + u) * _L, _L)] for u in range(4)]
        for u in range(4):
            acc[pl.ds((4 * i + u) * _L, _L)] = jnp.where(
                avs[u] == neg, 0.0, avs[u])
        return 0

    lax.fori_loop(0, slab // (4 * _L), fix_chunk, 0)
    pltpu.sync_copy(acc.at[pl.ds(0, slab)], out_hbm.at[pl.ds(base, slab)])


def _transpose_tc(x):
    def body(x_ref, o_ref):
        o_ref[...] = x_ref[...].T

    return pl.pallas_call(
        body,
        out_shape=jax.ShapeDtypeStruct((x.shape[1], x.shape[0]), jnp.float32),
    )(x)


def _layer_tc(aggT, hT, wl, bl, wr):
    def body(a_ref, h_ref, wl_ref, bl_ref, wr_ref, o_ref):
        o_ref[...] = (
            jnp.dot(wl_ref[...], a_ref[...], preferred_element_type=jnp.float32)
            + jnp.dot(wr_ref[...], h_ref[...], preferred_element_type=jnp.float32)
            + bl_ref[...])

    return pl.pallas_call(
        body, out_shape=jax.ShapeDtypeStruct(aggT.shape, jnp.float32),
    )(aggT, hT, wl, bl.reshape(-1, 1), wr)


def _head_tc(aggT, hT, wl, bl, wr, w1, b1, w2, b2, w3p, b3p):
    def body(a_ref, h_ref, wl_ref, bl_ref, wr_ref, w1_ref, b1_ref, w2_ref,
             b2_ref, w3_ref, b3_ref, o_ref):
        h3 = (jnp.dot(wl_ref[...], a_ref[...], preferred_element_type=jnp.float32)
              + jnp.dot(wr_ref[...], h_ref[...], preferred_element_type=jnp.float32)
              + bl_ref[...])
        t = jnp.maximum(
            jnp.dot(w1_ref[...], h3, preferred_element_type=jnp.float32)
            + b1_ref[...], 0.0)
        t = jnp.maximum(
            jnp.dot(w2_ref[...], t, preferred_element_type=jnp.float32)
            + b2_ref[...], 0.0)
        t = (jnp.dot(w3_ref[...], t, preferred_element_type=jnp.float32)
             + b3_ref[...])
        o_ref[...] = 1.0 / (1.0 + jnp.exp(-t))

    return pl.pallas_call(
        body,
        out_shape=jax.ShapeDtypeStruct((w3p.shape[0], aggT.shape[1]),
                                       jnp.float32),
    )(aggT, hT, wl, bl.reshape(-1, 1), wr, w1, b1.reshape(-1, 1), w2,
      b2.reshape(-1, 1), w3p, b3p.reshape(-1, 1))


def kernel(x, edge_index, batch, W1l, b1l, W1r, W2l, b2l, W2r, W3l, b3l, W3r,
           l1W, l1b, l2W, l2b, l3W, l3b):
    n, d = x.shape
    e = edge_index.shape[1]
    nc, ns = _sc_info()
    nw = nc * ns
    fpt = d // nw

    c = 2560
    while e % c:
        c //= 2

    mesh = plsc.VectorSubcoreMesh(core_axis_name="c", subcore_axis_name="s")

    segmax = pl.kernel(
        functools.partial(_segmax_t_body, nw, n, fpt, e, c),
        out_type=jax.ShapeDtypeStruct((d * n,), jnp.float32),
        mesh=mesh,
        scratch_types=[
            pltpu.VMEM((fpt * n,), jnp.float32),
            pltpu.VMEM((fpt * n + _L,), jnp.float32),
            pltpu.VMEM((n + _L,), jnp.int32),
            pltpu.VMEM((n + _L,), jnp.int32),
            pltpu.VMEM((c,), jnp.int32),
            pltpu.VMEM((c,), jnp.int32),
            pltpu.VMEM((c,), jnp.int32),
            pltpu.VMEM((c,), jnp.int32),
            pltpu.SemaphoreType.DMA,
            pltpu.SemaphoreType.DMA,
        ],
        compiler_params=pltpu.CompilerParams(needs_layout_passes=False),
    )

    def agg_of(hT):
        return segmax(edge_index, hT.reshape(-1)).reshape(d, n)

    w3p = jnp.zeros((8, l3W.shape[1]), jnp.float32).at[:l3W.shape[0]].set(l3W)
    b3p = jnp.zeros((8,), jnp.float32).at[:l3b.shape[0]].set(l3b)

    xT = _transpose_tc(x)
    h1T = _layer_tc(agg_of(xT), xT, W1l, b1l, W1r)
    h2T = _layer_tc(agg_of(h1T), h1T, W2l, b2l, W2r)
    outT = _head_tc(agg_of(h2T), h2T, W3l, b3l, W3r, l1W, l1b, l2W, l2b,
                    w3p, b3p)
    return outT[:l3W.shape[0]].T


# speedup vs baseline: 1.1777x; 1.1421x over previous
import functools

import jax
import jax.numpy as jnp
from jax import lax
from jax.experimental import pallas as pl
from jax.experimental.pallas import tpu as pltpu
from jax.experimental.pallas import tpu_sc as plsc

_L = 16


def _sc_info():
    try:
        info = plsc.get_sparse_core_info()
        return info.num_cores, info.num_subcores
    except Exception:
        return 2, 16


def _segmax_t_body(nw, w, fpt, e, c, ei_hbm, ht_hbm, out_hbm,
                   hts, acc, tmpa, tmpb, sa, da, sb, db, sema, semb):
    nc, _ = _sc_info()
    wid = lax.axis_index("s") * nc + lax.axis_index("c")
    slab = fpt * w
    base = wid * slab
    neg = jnp.full((_L,), -8323200, jnp.int32)
    lane = lax.iota(jnp.int32, _L)
    nch = e // c

    pltpu.sync_copy(ht_hbm.at[pl.ds(base, slab)], hts)

    def init_chunk(i, _):
        for u in range(4):
            acc[pl.ds((4 * i + u) * _L, _L)] = neg
        return 0

    lax.fori_loop(0, slab // (4 * _L), init_chunk, 0)

    def start(ci, s_buf, d_buf, sem):
        pltpu.async_copy(ei_hbm.at[0, pl.ds(ci * c, c)], s_buf, sem)
        pltpu.async_copy(ei_hbm.at[1, pl.ds(ci * c, c)], d_buf, sem)

    def wait(s_buf, d_buf, sem):
        pltpu.make_async_copy(ei_hbm.at[0, pl.ds(0, c)], s_buf, sem).wait()
        pltpu.make_async_copy(ei_hbm.at[0, pl.ds(0, c)], d_buf, sem).wait()

    def process(s_buf, d_buf):
        unroll = 4

        def round_(s16, d16, hvs, rem, tmp):
            idxw = jnp.where(rem, d16, w)
            plsc.store_scatter(tmp, [idxw], lane)
            back = plsc.load_gather(tmp, [idxw])
            winner = rem & (back == lane)
            idxus = [jnp.where(winner, d16 + f * w, slab) for f in range(fpt)]
            avs = [plsc.load_gather(acc, [idxus[f]]) for f in range(fpt)]
            mxs = [plsc.bitcast(
                jnp.maximum(plsc.bitcast(avs[f], jnp.bfloat16), hvs[f]),
                jnp.int32) for f in range(fpt)]
            for f in range(fpt):
                plsc.store_scatter(acc, [idxus[f]], mxs[f])
            return rem & (~winner)

        def group4(q, _):
            s16s = [s_buf[pl.ds((q * unroll + u) * _L, _L)]
                    for u in range(unroll)]
            d16s = [d_buf[pl.ds((q * unroll + u) * _L, _L)]
                    for u in range(unroll)]
            hvss = [[plsc.bitcast(plsc.load_gather(hts, [s16s[u] + f * w]),
                                  jnp.bfloat16)
                     for f in range(fpt)] for u in range(unroll)]
            rems = []
            for u in range(unroll):
                rems.append(round_(s16s[u], d16s[u], hvss[u],
                                   d16s[u] >= 0,
                                   tmpa if u % 2 == 0 else tmpb))

            anyrem = rems[0]
            for u in range(1, unroll):
                anyrem = anyrem | rems[u]

            @pl.when(jnp.any(anyrem))
            def _():
                for u in range(unroll):
                    gb = (q * unroll + u) * _L
                    s16 = s_buf[pl.ds(gb, _L)]
                    d16 = d_buf[pl.ds(gb, _L)]
                    hvs = [plsc.bitcast(
                        plsc.load_gather(hts, [s16 + f * w]), jnp.bfloat16)
                           for f in range(fpt)]

                    def cond(carry):
                        rem, r = carry
                        return jnp.any(rem) & (r < _L)

                    def wbody(carry):
                        rem, r = carry
                        return round_(s16, d16, hvs, rem, tmpa), r + 1

                    lax.while_loop(cond, wbody, (d16 >= 0, 0))

            return 0

        lax.fori_loop(0, c // (_L * unroll), group4, 0)

    start(0, sa, da, sema)

    def pairbody(p, _):
        start(2 * p + 1, sb, db, semb)
        wait(sa, da, sema)
        process(sa, da)
        start(2 * p + 2, sa, da, sema)
        wait(sb, db, semb)
        process(sb, db)
        return 0

    lax.fori_loop(0, (nch - 1) // 2, pairbody, 0)
    wait(sa, da, sema)
    process(sa, da)

    def fix_chunk(i, _):
        avs = [acc[pl.ds((4 * i + u) * _L, _L)] for u in range(4)]
        for u in range(4):
            acc[pl.ds((4 * i + u) * _L, _L)] = jnp.where(
                avs[u] == neg, 0, avs[u])
        return 0

    lax.fori_loop(0, slab // (4 * _L), fix_chunk, 0)
    pltpu.sync_copy(acc.at[pl.ds(0, slab)], out_hbm.at[pl.ds(base, slab)])


def _pack_rows(h):
    half = h.shape[0] // 2
    lo = lax.bitcast_convert_type(h[:half].astype(jnp.bfloat16),
                                  jnp.uint16).astype(jnp.uint32)
    hi = lax.bitcast_convert_type(h[half:].astype(jnp.bfloat16),
                                  jnp.uint16).astype(jnp.uint32)
    return lax.bitcast_convert_type(lo | (hi << 16), jnp.int32)


def _unpack_rows(p):
    pu = lax.bitcast_convert_type(p, jnp.uint32)
    lo = lax.bitcast_convert_type((pu & 0xFFFF).astype(jnp.uint16),
                                  jnp.bfloat16)
    hi = lax.bitcast_convert_type((pu >> 16).astype(jnp.uint16),
                                  jnp.bfloat16)
    return jnp.concatenate(
        [lo.astype(jnp.float32), hi.astype(jnp.float32)], axis=0)


def _transpose_tc(x):
    def body(x_ref, o_ref, p_ref):
        t = x_ref[...].T
        o_ref[...] = t
        p_ref[...] = _pack_rows(t)

    return pl.pallas_call(
        body,
        out_shape=(
            jax.ShapeDtypeStruct((x.shape[1], x.shape[0]), jnp.float32),
            jax.ShapeDtypeStruct((x.shape[1] // 2, x.shape[0]), jnp.int32),
        ),
    )(x)


def _layer_tc(aggP, hT, wl, bl, wr):
    def body(a_ref, h_ref, wl_ref, bl_ref, wr_ref, o_ref, p_ref):
        aggT = _unpack_rows(a_ref[...])
        o = (jnp.dot(wl_ref[...], aggT, preferred_element_type=jnp.float32)
             + jnp.dot(wr_ref[...], h_ref[...],
                       preferred_element_type=jnp.float32)
             + bl_ref[...])
        o_ref[...] = o
        p_ref[...] = _pack_rows(o)

    return pl.pallas_call(
        body,
        out_shape=(
            jax.ShapeDtypeStruct(hT.shape, jnp.float32),
            jax.ShapeDtypeStruct(aggP.shape, jnp.int32),
        ),
    )(aggP, hT, wl, bl.reshape(-1, 1), wr)


def _head_tc(aggP, hT, wl, bl, wr, w1, b1, w2, b2, w3p, b3p):
    def body(a_ref, h_ref, wl_ref, bl_ref, wr_ref, w1_ref, b1_ref, w2_ref,
             b2_ref, w3_ref, b3_ref, o_ref):
        aggT = _unpack_rows(a_ref[...])
        h3 = (jnp.dot(wl_ref[...], aggT, preferred_element_type=jnp.float32)
              + jnp.dot(wr_ref[...], h_ref[...], preferred_element_type=jnp.float32)
              + bl_ref[...])
        t = jnp.maximum(
            jnp.dot(w1_ref[...], h3, preferred_element_type=jnp.float32)
            + b1_ref[...], 0.0)
        t = jnp.maximum(
            jnp.dot(w2_ref[...], t, preferred_element_type=jnp.float32)
            + b2_ref[...], 0.0)
        t = (jnp.dot(w3_ref[...], t, preferred_element_type=jnp.float32)
             + b3_ref[...])
        o_ref[...] = 1.0 / (1.0 + jnp.exp(-t))

    return pl.pallas_call(
        body,
        out_shape=jax.ShapeDtypeStruct((w3p.shape[0], hT.shape[1]),
                                       jnp.float32),
    )(aggP, hT, wl, bl.reshape(-1, 1), wr, w1, b1.reshape(-1, 1), w2,
      b2.reshape(-1, 1), w3p, b3p.reshape(-1, 1))


def kernel(x, edge_index, batch, W1l, b1l, W1r, W2l, b2l, W2r, W3l, b3l, W3r,
           l1W, l1b, l2W, l2b, l3W, l3b):
    n, d = x.shape
    e = edge_index.shape[1]
    nc, ns = _sc_info()
    nw = nc * ns
    dp = d // 2
    fpt = dp // nw

    c = 2560
    while e % c:
        c //= 2

    mesh = plsc.VectorSubcoreMesh(core_axis_name="c", subcore_axis_name="s")

    segmax = pl.kernel(
        functools.partial(_segmax_t_body, nw, n, fpt, e, c),
        out_type=jax.ShapeDtypeStruct((dp * n,), jnp.int32),
        mesh=mesh,
        scratch_types=[
            pltpu.VMEM((fpt * n,), jnp.int32),
            pltpu.VMEM((fpt * n + _L,), jnp.int32),
            pltpu.VMEM((n + _L,), jnp.int32),
            pltpu.VMEM((n + _L,), jnp.int32),
            pltpu.VMEM((c,), jnp.int32),
            pltpu.VMEM((c,), jnp.int32),
            pltpu.VMEM((c,), jnp.int32),
            pltpu.VMEM((c,), jnp.int32),
            pltpu.SemaphoreType.DMA,
            pltpu.SemaphoreType.DMA,
        ],
        compiler_params=pltpu.CompilerParams(needs_layout_passes=False),
    )

    def agg_of(hTp):
        return segmax(edge_index, hTp.reshape(-1)).reshape(dp, n)

    w3p = jnp.zeros((8, l3W.shape[1]), jnp.float32).at[:l3W.shape[0]].set(l3W)
    b3p = jnp.zeros((8,), jnp.float32).at[:l3b.shape[0]].set(l3b)

    xT, xTp = _transpose_tc(x)
    h1T, h1Tp = _layer_tc(agg_of(xTp), xT, W1l, b1l, W1r)
    h2T, h2Tp = _layer_tc(agg_of(h1Tp), h1T, W2l, b2l, W2r)
    outT = _head_tc(agg_of(h2Tp), h2T, W3l, b3l, W3r, l1W, l1b, l2W, l2b,
                    w3p, b3p)
    return outT[:l3W.shape[0]].T
